# baseline (speedup 1.0000x reference)
import functools

import jax
import jax.numpy as jnp
from jax import lax
from jax.experimental import pallas as pl
from jax.experimental.pallas import tpu as pltpu
from jax.experimental.pallas import tpu_sc as plsc

NU = 5000
NI = 5000
D = 128
P = 5120
NP = 2 * P
NL = 4
E = 160000
EPC = 64
CAP = 16384
RPT = 320
RB = 32
NSUB = 16
NBUF = 4
LEAD = 2
SHIFT = 14
SCH = 2000

_mesh = plsc.VectorSubcoreMesh(core_axis_name="c", subcore_axis_name="s")


def _hsum(x):
    idx = lax.iota(jnp.int32, 16)
    for sh in (8, 4, 2, 1):
        perm = lax.bitwise_and(idx + sh, 15)
        x = x + x.at[perm].get(mode="promise_in_bounds")
    return x


def _nrsqrt(p):
    ip = lax.bitcast_convert_type(p, jnp.int32)
    iy = jnp.full((16,), 0x5F3759DF, jnp.int32) - \
        lax.shift_right_arithmetic(ip, jnp.full((16,), 1, jnp.int32))
    y = lax.bitcast_convert_type(iy, jnp.float32)
    for _ in range(3):
        y = y * (jnp.float32(1.5) - jnp.float32(0.5) * p * y * y)
    return y


def _splat16(v):
    return jnp.full((16,), v, jnp.int32)


def _layer_body(xs_in, ego, elists, counts, dnode, acc_in, xs_out, acc_out,
                acc_l, pall, dib, cbuf,
                rbuf0, rbuf1, rbuf2, rbuf3,
                gsem0, gsem1, gsem2, gsem3):
    c = lax.axis_index("c")
    s = lax.axis_index("s")
    tile = c * NSUB + s

    def zrow(i, _):
        z = jnp.zeros((16,), jnp.float32)
        for r in range(8):
            acc_l[i, pl.ds(16 * r, 16)] = z
        return 0
    lax.fori_loop(0, RPT, zrow, 0)

    pltpu.sync_copy(elists.at[pl.ds(tile * CAP, CAP)], pall)
    pltpu.sync_copy(counts.at[tile], cbuf)
    cnt = cbuf[pl.ds(0, 16)][0]
    nch = (cnt + (EPC - 1)) // EPC

    rbufs = (rbuf0, rbuf1, rbuf2, rbuf3)
    gsems = (gsem0, gsem1, gsem2, gsem3)
    m14 = _splat16((1 << SHIFT) - 1)
    s14 = _splat16(SHIFT)

    def unpack_dst(k, b):
        for g in range(EPC // 16):
            w = pall[pl.ds(k * EPC + 16 * g, 16)]
            dib[b, pl.ds(16 * g, 16)] = lax.bitwise_and(w, m14)

    for b in range(LEAD):
        @pl.when(b < nch)
        def _():
            unpack_dst(b, b)
            pltpu.async_copy(xs_in.at[dib.at[b]], rbufs[b], gsems[b])

    def step(q, _):
        for b in range(NBUF):
            k = NBUF * q + b
            b2 = (b + LEAD) % NBUF

            @pl.when(k + LEAD < nch)
            def _():
                unpack_dst(k + LEAD, b2)
                pltpu.async_copy(xs_in.at[dib.at[b2]], rbufs[b2], gsems[b2])

            @pl.when(k < nch)
            def _():
                pltpu.make_async_copy(
                    xs_in.at[dib.at[b]], rbufs[b], gsems[b]).wait()

                def grp(g, _):
                    w = pall[pl.ds(k * EPC + 16 * g, 16)]
                    src16 = lax.shift_right_logical(w, s14)
                    for l in range(16):
                        j16 = 16 * g + l
                        sl = src16[l]
                        for r in range(8):
                            plsc.addupdate(
                                acc_l.at[sl, pl.ds(16 * r, 16)],
                                rbufs[b][j16, pl.ds(16 * r, 16)])
                    return 0
                lax.fori_loop(0, EPC // 16, grp, 0)
        return 0
    lax.fori_loop(0, (CAP // EPC) // NBUF, step, 0)

    def blk(b, _):
        loc = b * RB
        g = c * P + s * RPT + loc
        pltpu.sync_copy(ego.at[pl.ds(g, RB)], rbuf1.at[pl.ds(0, RB)])
        pltpu.sync_copy(acc_in.at[pl.ds(g, RB)], rbuf2.at[pl.ds(0, RB)])
        pltpu.sync_copy(dnode.at[pl.ds(g, RB)], rbuf3.at[0, pl.ds(32, RB)])
        for gg in range(RB // 16):
            dv = rbuf3[0, pl.ds(32 + 16 * gg, 16)]
            for l in range(16):
                rbuf3[16 * gg + l, pl.ds(0, 16)] = \
                    jnp.broadcast_to(dv[l], (16,))

        def row(i, _):
            di = rbuf3[i, pl.ds(0, 16)]
            dot = jnp.zeros((16,), jnp.float32)
            sa = jnp.zeros((16,), jnp.float32)
            se = jnp.zeros((16,), jnp.float32)
            for r in range(8):
                av = acc_l[loc + i, pl.ds(16 * r, 16)] * di
                ev = rbuf1[i, pl.ds(16 * r, 16)]
                acc_l[loc + i, pl.ds(16 * r, 16)] = av
                dot = dot + av * ev
                sa = sa + av * av
                se = se + ev * ev
            p = jnp.maximum(_hsum(sa) * _hsum(se), jnp.float32(1e-16))
            w = _hsum(dot) * _nrsqrt(p)
            for r in range(8):
                ov = acc_l[loc + i, pl.ds(16 * r, 16)] * w
                rbuf2[i, pl.ds(16 * r, 16)] = \
                    rbuf2[i, pl.ds(16 * r, 16)] + ov
                rbuf0[i, pl.ds(16 * r, 16)] = ov * di
            return 0
        lax.fori_loop(0, RB, row, 0)
        pltpu.sync_copy(rbuf0.at[pl.ds(0, RB)], xs_out.at[pl.ds(g, RB)])
        pltpu.sync_copy(rbuf2.at[pl.ds(0, RB)], acc_out.at[pl.ds(g, RB)])
        return 0
    lax.fori_loop(0, RPT // RB, blk, 0)


_layer = functools.partial(
    pl.kernel,
    out_type=(
        jax.ShapeDtypeStruct((NP, D), jnp.float32),
        jax.ShapeDtypeStruct((NP, D), jnp.float32),
    ),
    mesh=_mesh,
    scratch_types=[
        pltpu.VMEM((RPT, D), jnp.float32),
        pltpu.VMEM((CAP,), jnp.int32),
        pltpu.VMEM((NBUF, EPC), jnp.int32),
        pltpu.VMEM((16,), jnp.int32),
    ] + [pltpu.VMEM((EPC, D), jnp.float32)] * NBUF
      + [pltpu.SemaphoreType.DMA] * NBUF,
)(_layer_body)


def _pre_body(ego, deg, dnode_out, xs0_out, dbufd, dsplat, ebuf):
    c = lax.axis_index("c")
    s = lax.axis_index("s")

    def blk(b, _):
        loc = b * RB
        g = c * P + s * RPT + loc
        pltpu.sync_copy(ego.at[pl.ds(g, RB)], ebuf)
        pltpu.sync_copy(deg.at[pl.ds(g, RB)], dbufd)
        for gg in range(RB // 16):
            dd = _nrsqrt(dbufd[pl.ds(16 * gg, 16)] + jnp.float32(1e-07))
            dbufd[pl.ds(16 * gg, 16)] = dd
            for l in range(16):
                dsplat[16 * gg + l, :] = jnp.broadcast_to(dd[l], (16,))

        def row(i, _):
            di = dsplat[i]
            for r in range(8):
                ebuf[i, pl.ds(16 * r, 16)] = ebuf[i, pl.ds(16 * r, 16)] * di
            return 0
        lax.fori_loop(0, RB, row, 0)
        pltpu.sync_copy(dbufd, dnode_out.at[pl.ds(g, RB)])
        pltpu.sync_copy(ebuf, xs0_out.at[pl.ds(g, RB)])
        return 0
    lax.fori_loop(0, RPT // RB, blk, 0)


_pre = functools.partial(
    pl.kernel,
    out_type=(
        jax.ShapeDtypeStruct((NP,), jnp.float32),
        jax.ShapeDtypeStruct((NP, D), jnp.float32),
    ),
    mesh=_mesh,
    scratch_types=[
        pltpu.VMEM((RB,), jnp.float32),
        pltpu.VMEM((RB, 16), jnp.float32),
        pltpu.VMEM((RB, D), jnp.float32),
    ],
)(_pre_body)


def _bucket(w, src):
    order = jnp.argsort(src, stable=True)
    ws = w[order]
    ss = src[order]
    bounds = jnp.searchsorted(
        ss, jnp.arange(NSUB + 1, dtype=jnp.int32) * RPT).astype(jnp.int32)
    cnts = jnp.diff(bounds)
    offs = bounds[:-1]
    t = (jnp.arange(NSUB * CAP, dtype=jnp.int32) // CAP)
    q = jnp.arange(NSUB * CAP, dtype=jnp.int32) % CAP
    src_idx = jnp.clip(offs[t] + q, 0, E - 1)
    valid = q < cnts[t]
    padded = jnp.where(valid, ws[src_idx], jnp.int32(P - 1))
    padded = padded - ((t * RPT) << SHIFT) * valid.astype(jnp.int32)
    deg = jnp.diff(jnp.searchsorted(
        ss, jnp.arange(NU + 1, dtype=jnp.int32))).astype(jnp.float32)
    return padded, cnts, deg


def kernel(user_embeddings, item_embeddings, rows, cols):
    b0, c0, dg0 = _bucket((cols + P) | (rows << SHIFT), rows)
    b1, c1, dg1 = _bucket(rows | (cols << SHIFT), cols)
    elists = jnp.concatenate([b0, b1])
    counts = jnp.broadcast_to(
        jnp.concatenate([c0, c1])[:, None], (2 * NSUB, 16))
    deg = jnp.zeros((NP,), jnp.float32).at[:NU].set(dg0)
    deg = deg.at[P:P + NI].set(dg1)

    ego = jnp.zeros((NP, D), jnp.float32)
    ego = ego.at[:NU].set(user_embeddings).at[P:P + NI].set(item_embeddings)

    dnode, xs = _pre(ego, deg)
    acc = jnp.zeros((NP, D), jnp.float32)
    for _ in range(NL):
        xs, acc = _layer(xs, ego, elists, counts, dnode, acc)
    return (acc[:NU], acc[P:P + NI])


# speedup vs baseline: 4.5571x; 4.5571x over previous
import functools

import jax
import jax.numpy as jnp
from jax import lax
from jax.experimental import pallas as pl
from jax.experimental.pallas import tpu as pltpu
from jax.experimental.pallas import tpu_sc as plsc

NU = 5000
NI = 5000
D = 128
P = 5120
NP = 2 * P
NL = 4
E = 160000
EPT = 10240
EPC = 80
NCH = EPT // EPC
RPT = 320
RB = 32
NSUB = 16
NBUF = 4
LEAD = 2
SHIFT = 14

_mesh = plsc.VectorSubcoreMesh(core_axis_name="c", subcore_axis_name="s")


def _hsum(x):
    idx = lax.iota(jnp.int32, 16)
    for sh in (8, 4, 2, 1):
        perm = lax.bitwise_and(idx + sh, 15)
        x = x + x.at[perm].get(mode="promise_in_bounds")
    return x


def _nrsqrt(p):
    ip = lax.bitcast_convert_type(p, jnp.int32)
    iy = jnp.full((16,), 0x5F3759DF, jnp.int32) - \
        lax.shift_right_arithmetic(ip, jnp.full((16,), 1, jnp.int32))
    y = lax.bitcast_convert_type(iy, jnp.float32)
    for _ in range(3):
        y = y * (jnp.float32(1.5) - jnp.float32(0.5) * p * y * y)
    return y


def _unpack_dst(pall, k, ibuf, b):
    m = jnp.full((16,), (1 << SHIFT) - 1, jnp.int32)
    for g in range(EPC // 16):
        w = pall[k, pl.ds(16 * g, 16)]
        ibuf[b, pl.ds(16 * g, 16)] = lax.bitwise_and(w, m)


def _unpack_src(pall, k, ibuf, b):
    sh = jnp.full((16,), SHIFT, jnp.int32)
    for g in range(EPC // 16):
        w = pall[k, pl.ds(16 * g, 16)]
        ibuf[b, pl.ds(16 * g, 16)] = lax.shift_right_logical(w, sh)


def _layer_body(xs_in, ego, pidx, dnode, acc_in, xs_out, acc_out,
                acc_s, pall, dibuf, sibuf,
                rbuf0, rbuf1, rbuf2, rbuf3,
                gsem0, gsem1, gsem2, gsem3,
                ssem0, ssem1, ssem2, ssem3):
    c = lax.axis_index("c")
    s = lax.axis_index("s")

    def zrow(i, _):
        z = jnp.zeros((16,), jnp.float32)
        for r in range(8):
            rbuf0[i, pl.ds(16 * r, 16)] = z
        return 0
    lax.fori_loop(0, RB, zrow, 0)

    def zcp(b, _):
        pltpu.sync_copy(rbuf0.at[pl.ds(0, RB)],
                        acc_s.at[pl.ds(s * RPT + b * RB, RB)])
        return 0
    lax.fori_loop(0, RPT // RB, zcp, 0)

    tb = (c * NSUB + s) * NCH
    pltpu.sync_copy(pidx.at[pl.ds(tb, NCH)], pall)
    plsc.subcore_barrier()

    rbufs = (rbuf0, rbuf1, rbuf2, rbuf3)
    gsems = (gsem0, gsem1, gsem2, gsem3)
    ssems = (ssem0, ssem1, ssem2, ssem3)
    for b in range(LEAD):
        _unpack_dst(pall, b, dibuf, b)
        pltpu.async_copy(xs_in.at[dibuf.at[b]], rbufs[b], gsems[b])

    def step(q, _):
        for b in range(NBUF):
            k = NBUF * q + b
            b2 = (b + LEAD) % NBUF
            pltpu.make_async_copy(
                xs_in.at[dibuf.at[b]], rbufs[b], gsems[b]).wait()
            _unpack_src(pall, k, sibuf, b)
            pltpu.async_copy(rbufs[b], acc_s.at[sibuf.at[b]], ssems[b],
                             add=True)

            @pl.when(k + LEAD >= NBUF)
            def _():
                pltpu.make_async_copy(
                    rbufs[b2], acc_s.at[sibuf.at[b2]], ssems[b2]).wait()

            @pl.when(k + LEAD < NCH)
            def _():
                _unpack_dst(pall, k + LEAD, dibuf, b2)
                pltpu.async_copy(
                    xs_in.at[dibuf.at[b2]], rbufs[b2], gsems[b2])
        return 0
    lax.fori_loop(0, NCH // NBUF, step, 0)
    for b in range(LEAD, NBUF):
        pltpu.make_async_copy(
            rbufs[b], acc_s.at[sibuf.at[b]], ssems[b]).wait()
    plsc.subcore_barrier()

    def blk(b, _):
        loc = s * RPT + b * RB
        g = c * P + loc
        pltpu.sync_copy(acc_s.at[pl.ds(loc, RB)], rbuf0.at[pl.ds(0, RB)])
        pltpu.sync_copy(ego.at[pl.ds(g, RB)], rbuf1.at[pl.ds(0, RB)])
        pltpu.sync_copy(acc_in.at[pl.ds(g, RB)], rbuf2.at[pl.ds(0, RB)])
        pltpu.sync_copy(dnode.at[pl.ds(g, RB)], rbuf3.at[0, pl.ds(32, RB)])
        for gg in range(RB // 16):
            dv = rbuf3[0, pl.ds(32 + 16 * gg, 16)]
            for l in range(16):
                rbuf3[16 * gg + l, pl.ds(0, 16)] = \
                    jnp.broadcast_to(dv[l], (16,))

        def row(i, _):
            di = rbuf3[i, pl.ds(0, 16)]
            dot = jnp.zeros((16,), jnp.float32)
            sa = jnp.zeros((16,), jnp.float32)
            se = jnp.zeros((16,), jnp.float32)
            for r in range(8):
                av = rbuf0[i, pl.ds(16 * r, 16)] * di
                ev = rbuf1[i, pl.ds(16 * r, 16)]
                rbuf0[i, pl.ds(16 * r, 16)] = av
                dot = dot + av * ev
                sa = sa + av * av
                se = se + ev * ev
            p = jnp.maximum(_hsum(sa) * _hsum(se), jnp.float32(1e-16))
            w = _hsum(dot) * _nrsqrt(p)
            for r in range(8):
                ov = rbuf0[i, pl.ds(16 * r, 16)] * w
                rbuf2[i, pl.ds(16 * r, 16)] = \
                    rbuf2[i, pl.ds(16 * r, 16)] + ov
                rbuf0[i, pl.ds(16 * r, 16)] = ov * di
            return 0
        lax.fori_loop(0, RB, row, 0)
        pltpu.sync_copy(rbuf0.at[pl.ds(0, RB)], xs_out.at[pl.ds(g, RB)])
        pltpu.sync_copy(rbuf2.at[pl.ds(0, RB)], acc_out.at[pl.ds(g, RB)])
        return 0
    lax.fori_loop(0, RPT // RB, blk, 0)


_layer = functools.partial(
    pl.kernel,
    out_type=(
        jax.ShapeDtypeStruct((NP, D), jnp.float32),
        jax.ShapeDtypeStruct((NP, D), jnp.float32),
    ),
    mesh=_mesh,
    scratch_types=[
        pltpu.MemorySpace.VMEM_SHARED((P, D), jnp.float32),
        pltpu.VMEM((NCH, EPC), jnp.int32),
        pltpu.VMEM((NBUF, EPC), jnp.int32),
        pltpu.VMEM((NBUF, EPC), jnp.int32),
    ] + [pltpu.VMEM((EPC, D), jnp.float32)] * NBUF
      + [pltpu.SemaphoreType.DMA] * (2 * NBUF),
)(_layer_body)


def _pre_body(ego, pidx, dnode_out, xs0_out,
              deg_s, pall, sibuf, ones, dbufd, dsplat, ebuf):
    c = lax.axis_index("c")
    s = lax.axis_index("s")
    for gg in range(RB // 16):
        dbufd[pl.ds(16 * gg, 16)] = jnp.zeros((16,), jnp.float32)

    def zcp(b, _):
        pltpu.sync_copy(dbufd, deg_s.at[pl.ds(s * RPT + b * RB, RB)])
        return 0
    lax.fori_loop(0, RPT // RB, zcp, 0)
    for g in range(EPC // 16):
        ones[pl.ds(16 * g, 16)] = jnp.full((16,), 1.0, jnp.float32)
    tb = (c * NSUB + s) * NCH
    pltpu.sync_copy(pidx.at[pl.ds(tb, NCH)], pall)
    plsc.subcore_barrier()

    def step(k, _):
        _unpack_src(pall, k, sibuf, 0)
        pltpu.sync_copy(ones, deg_s.at[sibuf.at[0]], add=True)
        return 0
    lax.fori_loop(0, NCH, step, 0)
    plsc.subcore_barrier()

    def blk(b, _):
        loc = s * RPT + b * RB
        g = c * P + loc
        pltpu.sync_copy(deg_s.at[pl.ds(loc, RB)], dbufd)
        pltpu.sync_copy(ego.at[pl.ds(g, RB)], ebuf)
        for gg in range(RB // 16):
            dd = _nrsqrt(dbufd[pl.ds(16 * gg, 16)] + jnp.float32(1e-07))
            dbufd[pl.ds(16 * gg, 16)] = dd
            for l in range(16):
                dsplat[16 * gg + l, :] = jnp.broadcast_to(dd[l], (16,))

        def row(i, _):
            di = dsplat[i]
            for r in range(8):
                ebuf[i, pl.ds(16 * r, 16)] = ebuf[i, pl.ds(16 * r, 16)] * di
            return 0
        lax.fori_loop(0, RB, row, 0)
        pltpu.sync_copy(dbufd, dnode_out.at[pl.ds(g, RB)])
        pltpu.sync_copy(ebuf, xs0_out.at[pl.ds(g, RB)])
        return 0
    lax.fori_loop(0, RPT // RB, blk, 0)


_pre = functools.partial(
    pl.kernel,
    out_type=(
        jax.ShapeDtypeStruct((NP,), jnp.float32),
        jax.ShapeDtypeStruct((NP, D), jnp.float32),
    ),
    mesh=_mesh,
    scratch_types=[
        pltpu.MemorySpace.VMEM_SHARED((P,), jnp.float32),
        pltpu.VMEM((NCH, EPC), jnp.int32),
        pltpu.VMEM((1, EPC), jnp.int32),
        pltpu.VMEM((EPC,), jnp.float32),
        pltpu.VMEM((RB,), jnp.float32),
        pltpu.VMEM((RB, 16), jnp.float32),
        pltpu.VMEM((RB, D), jnp.float32),
    ],
)(_pre_body)


def _pad_side(a, fill):
    a = a.reshape(NSUB, E // NSUB)
    a = jnp.pad(a, ((0, 0), (0, EPT - E // NSUB)), constant_values=fill)
    return a.reshape(NSUB * NCH, EPC)


def kernel(user_embeddings, item_embeddings, rows, cols):
    dsti = jnp.concatenate([_pad_side(cols + P, 0), _pad_side(rows, 0)])
    srci = jnp.concatenate([_pad_side(rows, P - 1), _pad_side(cols, P - 1)])
    pidx = dsti | (srci << SHIFT)

    ego = jnp.zeros((NP, D), jnp.float32)
    ego = ego.at[:NU].set(user_embeddings).at[P:P + NI].set(item_embeddings)

    dnode, xs = _pre(ego, pidx)
    acc = jnp.zeros((NP, D), jnp.float32)
    for _ in range(NL):
        xs, acc = _layer(xs, ego, pidx, dnode, acc)
    return (acc[:NU], acc[P:P + NI])


# speedup vs baseline: 5.0435x; 1.1067x over previous
import functools

import jax
import jax.numpy as jnp
from jax import lax
from jax.experimental import pallas as pl
from jax.experimental.pallas import tpu as pltpu
from jax.experimental.pallas import tpu_sc as plsc

NU = 5000
NI = 5000
D = 128
DW = D // 2
P = 5120
NP = 2 * P
NL = 4
E = 160000
EPT = 10240
EPC = 64
NCH = EPT // EPC
RPT = 320
RB = 32
NSUB = 16
NBUF = 4
LEAD = 2
SHIFT = 14

_mesh = plsc.VectorSubcoreMesh(core_axis_name="c", subcore_axis_name="s")


def _splat(v):
    return jnp.full((16,), v, jnp.int32)


def _hsum(x):
    idx = lax.iota(jnp.int32, 16)
    for sh in (8, 4, 2, 1):
        perm = lax.bitwise_and(idx + sh, 15)
        x = x + x.at[perm].get(mode="promise_in_bounds")
    return x


def _nrsqrt(p):
    ip = lax.bitcast_convert_type(p, jnp.int32)
    iy = _splat(0x5F3759DF) - lax.shift_right_arithmetic(ip, _splat(1))
    y = lax.bitcast_convert_type(iy, jnp.float32)
    for _ in range(3):
        y = y * (jnp.float32(1.5) - jnp.float32(0.5) * p * y * y)
    return y


def _pack_pair(lo, hi):
    il = lax.bitcast_convert_type(lo, jnp.int32) + _splat(0x8000)
    ih = lax.bitcast_convert_type(hi, jnp.int32) + _splat(0x8000)
    return lax.bitwise_or(
        lax.shift_right_logical(il, _splat(16)),
        lax.bitwise_and(ih, _splat(-65536)))


def _unpack_dst(pall, k, ibuf, b):
    m = _splat((1 << SHIFT) - 1)
    for g in range(EPC // 16):
        w = pall[k, pl.ds(16 * g, 16)]
        ibuf[b, pl.ds(16 * g, 16)] = lax.bitwise_and(w, m)


def _unpack_src(pall, k, ibuf, b):
    sh = _splat(SHIFT)
    for g in range(EPC // 16):
        w = pall[k, pl.ds(16 * g, 16)]
        ibuf[b, pl.ds(16 * g, 16)] = lax.shift_right_logical(w, sh)


def _layer_body(xs_in, ego, pidx, dnode, acc_in, xs_out, acc_out,
                acc_s, pall, dibuf, sibuf,
                rbuf0, rbuf1, rbuf2, rbuf3,
                sbuf0, sbuf1, sbuf2, sbuf3,
                gsem0, gsem1, gsem2, gsem3,
                ssem0, ssem1, ssem2, ssem3):
    c = lax.axis_index("c")
    s = lax.axis_index("s")

    def zrow(i, _):
        z = jnp.zeros((16,), jnp.float32)
        for r in range(8):
            sbuf0[i, pl.ds(16 * r, 16)] = z
        return 0
    lax.fori_loop(0, RB, zrow, 0)

    def zcp(b, _):
        pltpu.sync_copy(sbuf0.at[pl.ds(0, RB)],
                        acc_s.at[pl.ds(s * RPT + b * RB, RB)])
        return 0
    lax.fori_loop(0, RPT // RB, zcp, 0)

    tb = (c * NSUB + s) * NCH
    pltpu.sync_copy(pidx.at[pl.ds(tb, NCH)], pall)
    plsc.subcore_barrier()

    rbufs = (rbuf0, rbuf1, rbuf2, rbuf3)
    sbufs = (sbuf0, sbuf1, sbuf2, sbuf3)
    gsems = (gsem0, gsem1, gsem2, gsem3)
    ssems = (ssem0, ssem1, ssem2, ssem3)
    for b in range(LEAD):
        _unpack_dst(pall, b, dibuf, b)
        pltpu.async_copy(xs_in.at[dibuf.at[b]], rbufs[b], gsems[b])

    m16 = _splat(16)
    mhi = _splat(-65536)

    def step(q, _):
        for b in range(NBUF):
            k = NBUF * q + b
            b2 = (b + LEAD) % NBUF
            pltpu.make_async_copy(
                xs_in.at[dibuf.at[b]], rbufs[b], gsems[b]).wait()

            def crow(j, _):
                for g in range(DW // 16):
                    w = rbufs[b][j, pl.ds(16 * g, 16)]
                    lo = lax.bitcast_convert_type(
                        lax.shift_left(w, m16), jnp.float32)
                    hi = lax.bitcast_convert_type(
                        lax.bitwise_and(w, mhi), jnp.float32)
                    sbufs[b][j, pl.ds(32 * g, 16)] = lo
                    sbufs[b][j, pl.ds(32 * g + 16, 16)] = hi
                return 0
            lax.fori_loop(0, EPC, crow, 0)

            _unpack_src(pall, k, sibuf, b)
            pltpu.async_copy(sbufs[b], acc_s.at[sibuf.at[b]], ssems[b],
                             add=True)

            @pl.when(k + LEAD >= NBUF)
            def _():
                pltpu.make_async_copy(
                    sbufs[b2], acc_s.at[sibuf.at[b2]], ssems[b2]).wait()

            @pl.when(k + LEAD < NCH)
            def _():
                _unpack_dst(pall, k + LEAD, dibuf, b2)
                pltpu.async_copy(
                    xs_in.at[dibuf.at[b2]], rbufs[b2], gsems[b2])
        return 0
    lax.fori_loop(0, NCH // NBUF, step, 0)
    for b in range(LEAD, NBUF):
        pltpu.make_async_copy(
            sbufs[b], acc_s.at[sibuf.at[b]], ssems[b]).wait()
    plsc.subcore_barrier()

    def blk(b, _):
        loc = s * RPT + b * RB
        g = c * P + loc
        pltpu.sync_copy(acc_s.at[pl.ds(loc, RB)], sbuf0.at[pl.ds(0, RB)])
        pltpu.sync_copy(ego.at[pl.ds(g, RB)], sbuf1.at[pl.ds(0, RB)])
        pltpu.sync_copy(acc_in.at[pl.ds(g, RB)], sbuf2.at[pl.ds(0, RB)])
        pltpu.sync_copy(dnode.at[pl.ds(g, RB)], sbuf3.at[0, pl.ds(32, RB)])
        for gg in range(RB // 16):
            dv = sbuf3[0, pl.ds(32 + 16 * gg, 16)]
            for l in range(16):
                sbuf3[16 * gg + l, pl.ds(0, 16)] = \
                    jnp.broadcast_to(dv[l], (16,))

        def row(i, _):
            di = sbuf3[i, pl.ds(0, 16)]
            dot = jnp.zeros((16,), jnp.float32)
            sa = jnp.zeros((16,), jnp.float32)
            se = jnp.zeros((16,), jnp.float32)
            for r in range(8):
                av = sbuf0[i, pl.ds(16 * r, 16)] * di
                ev = sbuf1[i, pl.ds(16 * r, 16)]
                sbuf0[i, pl.ds(16 * r, 16)] = av
                dot = dot + av * ev
                sa = sa + av * av
                se = se + ev * ev
            p = jnp.maximum(_hsum(sa) * _hsum(se), jnp.float32(1e-16))
            w = _hsum(dot) * _nrsqrt(p)
            lo = None
            for r in range(8):
                ov = sbuf0[i, pl.ds(16 * r, 16)] * w
                sbuf2[i, pl.ds(16 * r, 16)] = \
                    sbuf2[i, pl.ds(16 * r, 16)] + ov
                ovd = ov * di
                if r % 2 == 0:
                    lo = ovd
                else:
                    rbuf0[i, pl.ds(16 * (r // 2), 16)] = _pack_pair(lo, ovd)
            return 0
        lax.fori_loop(0, RB, row, 0)
        pltpu.sync_copy(rbuf0.at[pl.ds(0, RB)], xs_out.at[pl.ds(g, RB)])
        pltpu.sync_copy(sbuf2.at[pl.ds(0, RB)], acc_out.at[pl.ds(g, RB)])
        return 0
    lax.fori_loop(0, RPT // RB, blk, 0)


_layer = functools.partial(
    pl.kernel,
    out_type=(
        jax.ShapeDtypeStruct((NP, DW), jnp.int32),
        jax.ShapeDtypeStruct((NP, D), jnp.float32),
    ),
    mesh=_mesh,
    scratch_types=[
        pltpu.MemorySpace.VMEM_SHARED((P, D), jnp.float32),
        pltpu.VMEM((NCH, EPC), jnp.int32),
        pltpu.VMEM((NBUF, EPC), jnp.int32),
        pltpu.VMEM((NBUF, EPC), jnp.int32),
    ] + [pltpu.VMEM((EPC, DW), jnp.int32)] * NBUF
      + [pltpu.VMEM((EPC, D), jnp.float32)] * NBUF
      + [pltpu.SemaphoreType.DMA] * (2 * NBUF),
    compiler_params=pltpu.CompilerParams(use_tc_tiling_on_sc=False),
)(_layer_body)


def _pre_body(ego, pidx, dnode_out, xs0_out,
              deg_s, pall, sibuf, ones, dbufd, dsplat, ebuf, pkbuf):
    c = lax.axis_index("c")
    s = lax.axis_index("s")
    for gg in range(RB // 16):
        dbufd[pl.ds(16 * gg, 16)] = jnp.zeros((16,), jnp.float32)

    def zcp(b, _):
        pltpu.sync_copy(dbufd, deg_s.at[pl.ds(s * RPT + b * RB, RB)])
        return 0
    lax.fori_loop(0, RPT // RB, zcp, 0)
    for g in range(EPC // 16):
        ones[pl.ds(16 * g, 16)] = jnp.full((16,), 1.0, jnp.float32)
    tb = (c * NSUB + s) * NCH
    pltpu.sync_copy(pidx.at[pl.ds(tb, NCH)], pall)
    plsc.subcore_barrier()

    def step(k, _):
        _unpack_src(pall, k, sibuf, 0)
        pltpu.sync_copy(ones, deg_s.at[sibuf.at[0]], add=True)
        return 0
    lax.fori_loop(0, NCH, step, 0)
    plsc.subcore_barrier()

    def blk(b, _):
        loc = s * RPT + b * RB
        g = c * P + loc
        pltpu.sync_copy(deg_s.at[pl.ds(loc, RB)], dbufd)
        pltpu.sync_copy(ego.at[pl.ds(g, RB)], ebuf)
        for gg in range(RB // 16):
            dd = _nrsqrt(dbufd[pl.ds(16 * gg, 16)] + jnp.float32(1e-07))
            dbufd[pl.ds(16 * gg, 16)] = dd
            for l in range(16):
                dsplat[16 * gg + l, :] = jnp.broadcast_to(dd[l], (16,))

        def row(i, _):
            di = dsplat[i]
            lo = None
            for r in range(8):
                xv = ebuf[i, pl.ds(16 * r, 16)] * di
                if r % 2 == 0:
                    lo = xv
                else:
                    pkbuf[i, pl.ds(16 * (r // 2), 16)] = _pack_pair(lo, xv)
            return 0
        lax.fori_loop(0, RB, row, 0)
        pltpu.sync_copy(dbufd, dnode_out.at[pl.ds(g, RB)])
        pltpu.sync_copy(pkbuf.at[pl.ds(0, RB)], xs0_out.at[pl.ds(g, RB)])
        return 0
    lax.fori_loop(0, RPT // RB, blk, 0)


_pre = functools.partial(
    pl.kernel,
    out_type=(
        jax.ShapeDtypeStruct((NP,), jnp.float32),
        jax.ShapeDtypeStruct((NP, DW), jnp.int32),
    ),
    mesh=_mesh,
    scratch_types=[
        pltpu.MemorySpace.VMEM_SHARED((P,), jnp.float32),
        pltpu.VMEM((NCH, EPC), jnp.int32),
        pltpu.VMEM((1, EPC), jnp.int32),
        pltpu.VMEM((EPC,), jnp.float32),
        pltpu.VMEM((RB,), jnp.float32),
        pltpu.VMEM((RB, 16), jnp.float32),
        pltpu.VMEM((RB, D), jnp.float32),
        pltpu.VMEM((RB, DW), jnp.int32),
    ],
    compiler_params=pltpu.CompilerParams(use_tc_tiling_on_sc=False),
)(_pre_body)


def _pad_side(a, fill):
    a = a.reshape(NSUB, E // NSUB)
    a = jnp.pad(a, ((0, 0), (0, EPT - E // NSUB)), constant_values=fill)
    return a.reshape(NSUB * NCH, EPC)


def kernel(user_embeddings, item_embeddings, rows, cols):
    dsti = jnp.concatenate([_pad_side(cols + P, 0), _pad_side(rows, 0)])
    srci = jnp.concatenate([_pad_side(rows, P - 1), _pad_side(cols, P - 1)])
    pidx = dsti | (srci << SHIFT)

    ego = jnp.zeros((NP, D), jnp.float32)
    ego = ego.at[:NU].set(user_embeddings).at[P:P + NI].set(item_embeddings)

    dnode, xs = _pre(ego, pidx)
    acc = jnp.zeros((NP, D), jnp.float32)
    for _ in range(NL):
        xs, acc = _layer(xs, ego, pidx, dnode, acc)
    return (acc[:NU], acc[P:P + NI])
